# baseline (speedup 1.0000x reference)
import numpy as np

import jax
import jax.numpy as jnp
from jax import lax
from jax.experimental import pallas as pl
from jax.experimental.pallas import tpu as pltpu
from jax.experimental.pallas import tpu_sc as plsc

NC = 2
NS = 16
L = 16
NW = NC * NS

H = 2048
W = 2048
PX = H * W
HALF_ROWS = H // 2
NWORDS_H = PX // 2 // 4
WPT = NWORDS_H // NW
CH = 8192
NCHUNK = WPT // CH

F255 = np.float32(255.0)
W0 = np.float32(0.299)
W1 = np.float32(0.587)
W2 = np.float32(0.114)
INV_EDGE = np.float32(256.0 / 255.0)
EDGE = np.float32(255.0 / 256.0)


def _bin_index(c0, c1, c2):
  g = (c0 * F255) * W0 + (c1 * F255) * W1
  g = g + (c2 * F255) * W2
  b = (g * INV_EDGE).astype(jnp.int32)
  bf = b.astype(jnp.float32)
  b = b - jnp.where(g < bf * EDGE, 1, 0)
  bf1 = (b + 1).astype(jnp.float32)
  b = b + jnp.where(g >= bf1 * EDGE, 1, 0)
  return jnp.minimum(b, 255)


BIN_ROWS = 256
QROWS = BIN_ROWS // 4


def _tc_bins_body(img_ref, out_ref):
  b = _bin_index(img_ref[0], img_ref[1], img_ref[2])
  w = b[0:QROWS]
  w = w | (b[QROWS:2 * QROWS] << 8)
  w = w | (b[2 * QROWS:3 * QROWS] << 16)
  w = w | (b[3 * QROWS:4 * QROWS] << 24)
  out_ref[...] = w


def _make_tc_bins(off_blocks):
  return pl.pallas_call(
      _tc_bins_body,
      grid=(HALF_ROWS // BIN_ROWS,),
      in_specs=[pl.BlockSpec((3, BIN_ROWS, W),
                             lambda i: (0, i + off_blocks, 0))],
      out_specs=pl.BlockSpec((QROWS, W), lambda i: (i, 0)),
      out_shape=jax.ShapeDtypeStruct((HALF_ROWS // 4, W), jnp.int32),
      compiler_params=pltpu.CompilerParams(
          dimension_semantics=("arbitrary",)),
  )


_tc_bins0 = _make_tc_bins(0)
_tc_bins1 = _make_tc_bins(HALF_ROWS // BIN_ROWS)


NBANK = 4


def _sc_hist_body(words, hist_out, buf0, buf1, hist_v, row_v, sem0, sem1):
  wid = lax.axis_index("s") * NC + lax.axis_index("c")
  base_w = wid * WPT

  zero16 = jnp.zeros((L,), jnp.float32)
  ones16 = jnp.ones((L,), jnp.float32)
  lane = lax.iota(jnp.int32, L)
  lane_bases = [lane * 256 + b * (L * 256) for b in range(NBANK)]

  def zero_body(i, c):
    hist_v[pl.ds(i * L, L)] = zero16
    return c

  lax.fori_loop(0, NBANK * 256, zero_body, 0)

  bufs = (buf0, buf1)
  sems = (sem0, sem1)

  def start_chunk(k):
    start = base_w + k * CH
    return pltpu.async_copy(words.at[pl.ds(start, CH)], bufs[k % 2],
                            sems[k % 2])

  def do_chunk(buf):
    @plsc.parallel_loop(0, CH // L, step=1, unroll=4)
    def body(i):
      w = buf[pl.ds(i * L, L)]
      b0 = w & 255
      b1 = lax.shift_right_logical(w, 8) & 255
      b2 = lax.shift_right_logical(w, 16) & 255
      b3 = lax.shift_right_logical(w, 24)
      plsc.addupdate_scatter(hist_v, [lane_bases[0] | b0], ones16)
      plsc.addupdate_scatter(hist_v, [lane_bases[1] | b1], ones16)
      plsc.addupdate_scatter(hist_v, [lane_bases[2] | b2], ones16)
      plsc.addupdate_scatter(hist_v, [lane_bases[3] | b3], ones16)

  cps = [None, None]
  cps[0] = start_chunk(0)
  for k in range(NCHUNK):
    if k + 1 < NCHUNK:
      cps[(k + 1) % 2] = start_chunk(k + 1)
    cps[k % 2].wait()
    do_chunk(bufs[k % 2])

  for c in range(16):
    def fold(l, acc):
      return acc + hist_v[pl.ds(l * 256 + c * L, L)]
    row_v[pl.ds(c * L, L)] = lax.fori_loop(0, NBANK * 16, fold, zero16)

  pltpu.sync_copy(row_v, hist_out.at[pl.ds(wid * 256, 256)])


_sc_hist = pl.kernel(
    _sc_hist_body,
    out_type=jax.ShapeDtypeStruct((NW * 256,), jnp.float32),
    mesh=plsc.VectorSubcoreMesh(core_axis_name="c", subcore_axis_name="s"),
    scratch_types=[
        pltpu.VMEM((CH,), jnp.int32),
        pltpu.VMEM((CH,), jnp.int32),
        pltpu.VMEM((NBANK * L * 256,), jnp.float32),
        pltpu.VMEM((256,), jnp.float32),
        pltpu.SemaphoreType.DMA,
        pltpu.SemaphoreType.DMA,
    ],
    compiler_params=pltpu.CompilerParams(use_tc_tiling_on_sc=False,
                                         needs_layout_passes=False),
)


def _tc_rescale_body(hist0_ref, hist1_ref, img_ref, out_ref, ab_ref):
  i = pl.program_id(0)

  @pl.when(i == 0)
  def _():
    parts = hist0_ref[...] + hist1_ref[...]
    acc = jnp.sum(parts, axis=0, keepdims=True)
    lanes = lax.broadcasted_iota(jnp.int32, (1, 256), 1)
    sh = 1
    for _ in range(8):
      rolled = pltpu.roll(acc, sh, axis=1)
      acc = acc + jnp.where(lanes >= sh, rolled, jnp.float32(0.0))
      sh *= 2
    maximum = jnp.max(acc)
    clip_value = (maximum / jnp.float32(100.0)) / jnp.float32(2.0)
    min_g = jnp.sum((acc < clip_value).astype(jnp.float32))
    max_g = jnp.float32(255.0) - jnp.sum(
        (acc >= maximum - clip_value).astype(jnp.float32))
    diff = max_g - min_g
    deg = max_g <= min_g
    safe = jnp.where(deg, jnp.float32(1.0), diff)
    alpha = jnp.where(deg, jnp.float32(1.0), jnp.float32(1.0) / safe)
    beta = jnp.where(deg, jnp.float32(0.0), (-min_g) / safe)
    ab_ref[0] = alpha
    ab_ref[1] = beta

  a = ab_ref[0]
  b = ab_ref[1]
  out_ref[...] = jnp.clip(img_ref[...] * a + b, 0.0, 1.0)


ROWS_PER_BLOCK = 512
NBLOCKS = (3 * H) // ROWS_PER_BLOCK

_tc_rescale = pl.pallas_call(
    _tc_rescale_body,
    grid=(NBLOCKS,),
    in_specs=[
        pl.BlockSpec((NW, 256), lambda i: (0, 0)),
        pl.BlockSpec((NW, 256), lambda i: (0, 0)),
        pl.BlockSpec((ROWS_PER_BLOCK, W), lambda i: (i, 0)),
    ],
    out_specs=pl.BlockSpec((ROWS_PER_BLOCK, W), lambda i: (i, 0)),
    out_shape=jax.ShapeDtypeStruct((3 * H, W), jnp.float32),
    scratch_shapes=[pltpu.SMEM((2,), jnp.float32)],
    compiler_params=pltpu.CompilerParams(
        dimension_semantics=("arbitrary",)),
)


@jax.jit
def kernel(image_tensor):
  p0 = _tc_bins0(image_tensor)
  h0 = _sc_hist(p0.reshape(NWORDS_H))
  p1 = _tc_bins1(image_tensor)
  h1 = _sc_hist(p1.reshape(NWORDS_H))
  out = _tc_rescale(h0.reshape(NW, 256), h1.reshape(NW, 256),
                    image_tensor.reshape(3 * H, W))
  return out.reshape(3, H, W)


# speedup vs baseline: 1.1167x; 1.1167x over previous
import numpy as np

import jax
import jax.numpy as jnp
from jax import lax
from jax.experimental import pallas as pl
from jax.experimental.pallas import tpu as pltpu
from jax.experimental.pallas import tpu_sc as plsc

NC = 2
NS = 16
L = 16
NW = NC * NS

H = 2048
W = 2048
PX = H * W
NWORDS = PX // 4
WPT = NWORDS // NW
CH = 8192
NCHUNK = WPT // CH

F255 = np.float32(255.0)
W0 = np.float32(0.299)
W1 = np.float32(0.587)
W2 = np.float32(0.114)
INV_EDGE = np.float32(256.0 / 255.0)
EDGE = np.float32(255.0 / 256.0)


def _bin_index(c0, c1, c2):
  g = (c0 * F255) * W0 + (c1 * F255) * W1
  g = g + (c2 * F255) * W2
  b = (g * INV_EDGE).astype(jnp.int32)
  bf = b.astype(jnp.float32)
  b = b - jnp.where(g < bf * EDGE, 1, 0)
  bf1 = (b + 1).astype(jnp.float32)
  b = b + jnp.where(g >= bf1 * EDGE, 1, 0)
  return jnp.minimum(b, 255)


BIN_ROWS = 256
QROWS = BIN_ROWS // 4


def _tc_bins_body(img_ref, out_ref):
  b = _bin_index(img_ref[0], img_ref[1], img_ref[2])
  w = b[0:QROWS]
  w = w | (b[QROWS:2 * QROWS] << 8)
  w = w | (b[2 * QROWS:3 * QROWS] << 16)
  w = w | (b[3 * QROWS:4 * QROWS] << 24)
  out_ref[...] = w


_tc_bins = pl.pallas_call(
    _tc_bins_body,
    grid=(H // BIN_ROWS,),
    in_specs=[pl.BlockSpec((3, BIN_ROWS, W), lambda i: (0, i, 0))],
    out_specs=pl.BlockSpec((QROWS, W), lambda i: (i, 0)),
    out_shape=jax.ShapeDtypeStruct((H // 4, W), jnp.int32),
    compiler_params=pltpu.CompilerParams(
        dimension_semantics=("arbitrary",)),
)


NBANK = 4


def _sc_hist_body(words, hist_out, buf0, buf1, hist_v, row_v, sem0, sem1):
  wid = lax.axis_index("s") * NC + lax.axis_index("c")
  base_w = wid * WPT

  zero16 = jnp.zeros((L,), jnp.float32)
  ones16 = jnp.ones((L,), jnp.float32)
  lane = lax.iota(jnp.int32, L)
  lane_bases = [lane * 256 + b * (L * 256) for b in range(NBANK)]

  @plsc.parallel_loop(0, NBANK * 256, step=1, unroll=4)
  def zero_body(i):
    hist_v[pl.ds(i * L, L)] = zero16

  for c in range(16):
    row_v[pl.ds(c * L, L)] = zero16

  bufs = (buf0, buf1)
  sems = (sem0, sem1)

  def start_chunk(k):
    start = base_w + k * CH
    return pltpu.async_copy(words.at[pl.ds(start, CH)], bufs[k % 2],
                            sems[k % 2])

  def do_chunk(buf):
    @plsc.parallel_loop(0, CH // L, step=1, unroll=4)
    def body(i):
      w = buf[pl.ds(i * L, L)]
      b0 = w & 255
      b1 = lax.shift_right_logical(w, 8) & 255
      b2 = lax.shift_right_logical(w, 16) & 255
      b3 = lax.shift_right_logical(w, 24)
      plsc.addupdate_scatter(hist_v, [lane_bases[0] | b0], ones16)
      plsc.addupdate_scatter(hist_v, [lane_bases[1] | b1], ones16)
      plsc.addupdate_scatter(hist_v, [lane_bases[2] | b2], ones16)
      plsc.addupdate_scatter(hist_v, [lane_bases[3] | b3], ones16)

  cps = [None, None]
  cps[0] = start_chunk(0)
  for k in range(NCHUNK):
    if k + 1 < NCHUNK:
      cps[(k + 1) % 2] = start_chunk(k + 1)
    cps[k % 2].wait()
    do_chunk(bufs[k % 2])

  @plsc.parallel_loop(0, NBANK * 16 * 16, step=1, unroll=4)
  def fold_body(j):
    v = hist_v[pl.ds(j * L, L)]
    plsc.addupdate_scatter(row_v, [lane + (j & 15) * L], v)

  pltpu.sync_copy(row_v, hist_out.at[pl.ds(wid * 256, 256)])


_sc_hist = pl.kernel(
    _sc_hist_body,
    out_type=jax.ShapeDtypeStruct((NW * 256,), jnp.float32),
    mesh=plsc.VectorSubcoreMesh(core_axis_name="c", subcore_axis_name="s"),
    scratch_types=[
        pltpu.VMEM((CH,), jnp.int32),
        pltpu.VMEM((CH,), jnp.int32),
        pltpu.VMEM((NBANK * L * 256,), jnp.float32),
        pltpu.VMEM((256,), jnp.float32),
        pltpu.SemaphoreType.DMA,
        pltpu.SemaphoreType.DMA,
    ],
    compiler_params=pltpu.CompilerParams(use_tc_tiling_on_sc=False,
                                         needs_layout_passes=False),
)


def _tc_rescale_body(hist_ref, img_ref, out_ref, ab_ref):
  i = pl.program_id(0)

  @pl.when(i == 0)
  def _():
    parts = hist_ref[...]
    acc = jnp.sum(parts, axis=0, keepdims=True)
    lanes = lax.broadcasted_iota(jnp.int32, (1, 256), 1)
    sh = 1
    for _ in range(8):
      rolled = pltpu.roll(acc, sh, axis=1)
      acc = acc + jnp.where(lanes >= sh, rolled, jnp.float32(0.0))
      sh *= 2
    maximum = jnp.max(acc)
    clip_value = (maximum / jnp.float32(100.0)) / jnp.float32(2.0)
    min_g = jnp.sum((acc < clip_value).astype(jnp.float32))
    max_g = jnp.float32(255.0) - jnp.sum(
        (acc >= maximum - clip_value).astype(jnp.float32))
    diff = max_g - min_g
    deg = max_g <= min_g
    safe = jnp.where(deg, jnp.float32(1.0), diff)
    alpha = jnp.where(deg, jnp.float32(1.0), jnp.float32(1.0) / safe)
    beta = jnp.where(deg, jnp.float32(0.0), (-min_g) / safe)
    ab_ref[0] = alpha
    ab_ref[1] = beta

  a = ab_ref[0]
  b = ab_ref[1]
  out_ref[...] = jnp.clip(img_ref[...] * a + b, 0.0, 1.0)


ROWS_PER_BLOCK = 512
NBLOCKS = (3 * H) // ROWS_PER_BLOCK

_tc_rescale = pl.pallas_call(
    _tc_rescale_body,
    grid=(NBLOCKS,),
    in_specs=[
        pl.BlockSpec((NW, 256), lambda i: (0, 0)),
        pl.BlockSpec((ROWS_PER_BLOCK, W), lambda i: (i, 0)),
    ],
    out_specs=pl.BlockSpec((ROWS_PER_BLOCK, W), lambda i: (i, 0)),
    out_shape=jax.ShapeDtypeStruct((3 * H, W), jnp.float32),
    scratch_shapes=[pltpu.SMEM((2,), jnp.float32)],
    compiler_params=pltpu.CompilerParams(
        dimension_semantics=("arbitrary",)),
)


@jax.jit
def kernel(image_tensor):
  packed = _tc_bins(image_tensor)
  hist_parts = _sc_hist(packed.reshape(NWORDS))
  out = _tc_rescale(hist_parts.reshape(NW, 256),
                    image_tensor.reshape(3 * H, W))
  return out.reshape(3, H, W)


# speedup vs baseline: 1.1368x; 1.0180x over previous
import numpy as np

import jax
import jax.numpy as jnp
from jax import lax
from jax.experimental import pallas as pl
from jax.experimental.pallas import tpu as pltpu
from jax.experimental.pallas import tpu_sc as plsc

NC = 2
NS = 16
L = 16
NW = NC * NS

H = 2048
W = 2048
PX = H * W
NWORDS = PX // 4
WPT = NWORDS // NW
CH = 8192
NCHUNK = WPT // CH

F255 = np.float32(255.0)
W0 = np.float32(0.299)
W1 = np.float32(0.587)
W2 = np.float32(0.114)
INV_EDGE = np.float32(256.0 / 255.0)
EDGE = np.float32(255.0 / 256.0)


def _bin_index(c0, c1, c2):
  g = (c0 * F255) * W0 + (c1 * F255) * W1
  g = g + (c2 * F255) * W2
  b = (g * INV_EDGE).astype(jnp.int32)
  bf = b.astype(jnp.float32)
  b = b - jnp.where(g < bf * EDGE, 1, 0)
  bf1 = (b + 1).astype(jnp.float32)
  b = b + jnp.where(g >= bf1 * EDGE, 1, 0)
  return jnp.minimum(b, 255)


BIN_ROWS = 256
QROWS = BIN_ROWS // 4


def _tc_bins_body(img_ref, out_ref):
  b = _bin_index(img_ref[0], img_ref[1], img_ref[2])
  w = b[0:QROWS]
  w = w | (b[QROWS:2 * QROWS] << 8)
  w = w | (b[2 * QROWS:3 * QROWS] << 16)
  w = w | (b[3 * QROWS:4 * QROWS] << 24)
  out_ref[...] = w


_tc_bins = pl.pallas_call(
    _tc_bins_body,
    grid=(H // BIN_ROWS,),
    in_specs=[pl.BlockSpec((3, BIN_ROWS, W), lambda i: (0, i, 0))],
    out_specs=pl.BlockSpec((QROWS, W), lambda i: (i, 0)),
    out_shape=jax.ShapeDtypeStruct((H // 4, W), jnp.int32),
    compiler_params=pltpu.CompilerParams(
        dimension_semantics=("arbitrary",)),
)


NBANK = 4


def _sc_hist_body(words, hist_out, buf0, buf1, hist_v, row_v, sem0, sem1):
  wid = lax.axis_index("s") * NC + lax.axis_index("c")
  base_w = wid * WPT

  zero16 = jnp.zeros((L,), jnp.float32)
  ones16 = jnp.ones((L,), jnp.float32)
  lane = lax.iota(jnp.int32, L)
  lane_bases = [lane * 256 + b * (L * 256) for b in range(NBANK)]

  @plsc.parallel_loop(0, NBANK * 256, step=1, unroll=4)
  def zero_body(i):
    hist_v[pl.ds(i * L, L)] = zero16

  for c in range(16):
    row_v[pl.ds(c * L, L)] = zero16

  bufs = (buf0, buf1)
  sems = (sem0, sem1)

  def start_chunk(k):
    start = base_w + k * CH
    return pltpu.async_copy(words.at[pl.ds(start, CH)], bufs[k % 2],
                            sems[k % 2])

  def do_chunk(buf):
    @plsc.parallel_loop(0, CH // L, step=1, unroll=4)
    def body(i):
      w = buf[pl.ds(i * L, L)]
      b0 = w & 255
      b1 = lax.shift_right_logical(w, 8) & 255
      b2 = lax.shift_right_logical(w, 16) & 255
      b3 = lax.shift_right_logical(w, 24)
      plsc.addupdate_scatter(hist_v, [lane_bases[0] | b0], ones16)
      plsc.addupdate_scatter(hist_v, [lane_bases[1] | b1], ones16)
      plsc.addupdate_scatter(hist_v, [lane_bases[2] | b2], ones16)
      plsc.addupdate_scatter(hist_v, [lane_bases[3] | b3], ones16)

  cps = [None, None]
  cps[0] = start_chunk(0)
  for k in range(NCHUNK):
    if k + 1 < NCHUNK:
      cps[(k + 1) % 2] = start_chunk(k + 1)
    cps[k % 2].wait()
    do_chunk(bufs[k % 2])

  @plsc.parallel_loop(0, NBANK * 16 * 16, step=1, unroll=4)
  def fold_body(j):
    v = hist_v[pl.ds(j * L, L)]
    plsc.addupdate_scatter(row_v, [lane + (j & 15) * L], v)

  pltpu.sync_copy(row_v, hist_out.at[pl.ds(wid * 256, 256)])


_sc_hist = pl.kernel(
    _sc_hist_body,
    out_type=jax.ShapeDtypeStruct((NW * 256,), jnp.float32),
    mesh=plsc.VectorSubcoreMesh(core_axis_name="c", subcore_axis_name="s"),
    scratch_types=[
        pltpu.VMEM((CH,), jnp.int32),
        pltpu.VMEM((CH,), jnp.int32),
        pltpu.VMEM((NBANK * L * 256,), jnp.float32),
        pltpu.VMEM((256,), jnp.float32),
        pltpu.SemaphoreType.DMA,
        pltpu.SemaphoreType.DMA,
    ],
    compiler_params=pltpu.CompilerParams(use_tc_tiling_on_sc=False,
                                         needs_layout_passes=False),
)


def _tc_rescale_body(hist_ref, img_ref, out_ref, ab_ref):
  i = pl.program_id(0)

  @pl.when(i == 0)
  def _():
    parts = hist_ref[...]
    acc = jnp.sum(parts, axis=0, keepdims=True)
    lanes = lax.broadcasted_iota(jnp.int32, (1, 256), 1)
    sh = 1
    for _ in range(8):
      rolled = pltpu.roll(acc, sh, axis=1)
      acc = acc + jnp.where(lanes >= sh, rolled, jnp.float32(0.0))
      sh *= 2
    maximum = jnp.max(acc)
    clip_value = (maximum / jnp.float32(100.0)) / jnp.float32(2.0)
    min_g = jnp.sum((acc < clip_value).astype(jnp.float32))
    max_g = jnp.float32(255.0) - jnp.sum(
        (acc >= maximum - clip_value).astype(jnp.float32))
    diff = max_g - min_g
    deg = max_g <= min_g
    safe = jnp.where(deg, jnp.float32(1.0), diff)
    alpha = jnp.where(deg, jnp.float32(1.0), jnp.float32(1.0) / safe)
    beta = jnp.where(deg, jnp.float32(0.0), (-min_g) / safe)
    ab_ref[0] = alpha
    ab_ref[1] = beta

  a = ab_ref[0]
  b = ab_ref[1]
  out_ref[...] = jnp.clip(img_ref[...] * a + b, 0.0, 1.0)


ROWS_PER_BLOCK = 1024
NBLOCKS = (3 * H) // ROWS_PER_BLOCK

_tc_rescale = pl.pallas_call(
    _tc_rescale_body,
    grid=(NBLOCKS,),
    in_specs=[
        pl.BlockSpec((NW, 256), lambda i: (0, 0)),
        pl.BlockSpec((ROWS_PER_BLOCK, W), lambda i: (i, 0)),
    ],
    out_specs=pl.BlockSpec((ROWS_PER_BLOCK, W), lambda i: (i, 0)),
    out_shape=jax.ShapeDtypeStruct((3 * H, W), jnp.float32),
    scratch_shapes=[pltpu.SMEM((2,), jnp.float32)],
    compiler_params=pltpu.CompilerParams(
        dimension_semantics=("arbitrary",)),
)


@jax.jit
def kernel(image_tensor):
  packed = _tc_bins(image_tensor)
  hist_parts = _sc_hist(packed.reshape(NWORDS))
  out = _tc_rescale(hist_parts.reshape(NW, 256),
                    image_tensor.reshape(3 * H, W))
  return out.reshape(3, H, W)


# speedup vs baseline: 1.1404x; 1.0032x over previous
import numpy as np

import jax
import jax.numpy as jnp
from jax import lax
from jax.experimental import pallas as pl
from jax.experimental.pallas import tpu as pltpu
from jax.experimental.pallas import tpu_sc as plsc

NC = 2
NS = 16
L = 16
NW = NC * NS

H = 2048
W = 2048
PX = H * W
NWORDS = PX // 4
WPT = NWORDS // NW
CH = 8192
NCHUNK = WPT // CH

F255 = np.float32(255.0)
W0 = np.float32(0.299)
W1 = np.float32(0.587)
W2 = np.float32(0.114)
INV_EDGE = np.float32(256.0 / 255.0)
EDGE = np.float32(255.0 / 256.0)


def _bin_index(c0, c1, c2):
  g = (c0 * F255) * W0 + (c1 * F255) * W1
  g = g + (c2 * F255) * W2
  b = (g * INV_EDGE).astype(jnp.int32)
  bf = b.astype(jnp.float32)
  b = b - jnp.where(g < bf * EDGE, 1, 0)
  bf1 = (b + 1).astype(jnp.float32)
  b = b + jnp.where(g >= bf1 * EDGE, 1, 0)
  return jnp.minimum(b, 255)


BIN_ROWS = 512
QROWS = BIN_ROWS // 4


def _tc_bins_body(img_ref, out_ref):
  b = _bin_index(img_ref[0], img_ref[1], img_ref[2])
  w = b[0:QROWS]
  w = w | (b[QROWS:2 * QROWS] << 8)
  w = w | (b[2 * QROWS:3 * QROWS] << 16)
  w = w | (b[3 * QROWS:4 * QROWS] << 24)
  out_ref[...] = w


_tc_bins = pl.pallas_call(
    _tc_bins_body,
    grid=(H // BIN_ROWS,),
    in_specs=[pl.BlockSpec((3, BIN_ROWS, W), lambda i: (0, i, 0))],
    out_specs=pl.BlockSpec((QROWS, W), lambda i: (i, 0)),
    out_shape=jax.ShapeDtypeStruct((H // 4, W), jnp.int32),
    compiler_params=pltpu.CompilerParams(
        dimension_semantics=("arbitrary",)),
)


NBANK = 4


def _sc_hist_body(words, hist_out, buf0, buf1, hist_v, row_v, sem0, sem1):
  wid = lax.axis_index("s") * NC + lax.axis_index("c")
  base_w = wid * WPT

  zero16 = jnp.zeros((L,), jnp.float32)
  ones16 = jnp.ones((L,), jnp.float32)
  lane = lax.iota(jnp.int32, L)
  lane_bases = [lane * 256 + b * (L * 256) for b in range(NBANK)]

  @plsc.parallel_loop(0, NBANK * 256, step=1, unroll=4)
  def zero_body(i):
    hist_v[pl.ds(i * L, L)] = zero16

  for c in range(16):
    row_v[pl.ds(c * L, L)] = zero16

  bufs = (buf0, buf1)
  sems = (sem0, sem1)

  def start_chunk(k):
    start = base_w + k * CH
    return pltpu.async_copy(words.at[pl.ds(start, CH)], bufs[k % 2],
                            sems[k % 2])

  def do_chunk(buf):
    @plsc.parallel_loop(0, CH // L, step=1, unroll=4)
    def body(i):
      w = buf[pl.ds(i * L, L)]
      b0 = w & 255
      b1 = lax.shift_right_logical(w, 8) & 255
      b2 = lax.shift_right_logical(w, 16) & 255
      b3 = lax.shift_right_logical(w, 24)
      plsc.addupdate_scatter(hist_v, [lane_bases[0] | b0], ones16)
      plsc.addupdate_scatter(hist_v, [lane_bases[1] | b1], ones16)
      plsc.addupdate_scatter(hist_v, [lane_bases[2] | b2], ones16)
      plsc.addupdate_scatter(hist_v, [lane_bases[3] | b3], ones16)

  cps = [None, None]
  cps[0] = start_chunk(0)
  for k in range(NCHUNK):
    if k + 1 < NCHUNK:
      cps[(k + 1) % 2] = start_chunk(k + 1)
    cps[k % 2].wait()
    do_chunk(bufs[k % 2])

  @plsc.parallel_loop(0, NBANK * 16 * 16, step=1, unroll=4)
  def fold_body(j):
    v = hist_v[pl.ds(j * L, L)]
    plsc.addupdate_scatter(row_v, [lane + (j & 15) * L], v)

  pltpu.sync_copy(row_v, hist_out.at[pl.ds(wid * 256, 256)])


_sc_hist = pl.kernel(
    _sc_hist_body,
    out_type=jax.ShapeDtypeStruct((NW * 256,), jnp.float32),
    mesh=plsc.VectorSubcoreMesh(core_axis_name="c", subcore_axis_name="s"),
    scratch_types=[
        pltpu.VMEM((CH,), jnp.int32),
        pltpu.VMEM((CH,), jnp.int32),
        pltpu.VMEM((NBANK * L * 256,), jnp.float32),
        pltpu.VMEM((256,), jnp.float32),
        pltpu.SemaphoreType.DMA,
        pltpu.SemaphoreType.DMA,
    ],
    compiler_params=pltpu.CompilerParams(use_tc_tiling_on_sc=False,
                                         needs_layout_passes=False),
)


def _tc_rescale_body(hist_ref, img_ref, out_ref, ab_ref):
  i = pl.program_id(0)

  @pl.when(i == 0)
  def _():
    parts = hist_ref[...]
    acc = jnp.sum(parts, axis=0, keepdims=True)
    lanes = lax.broadcasted_iota(jnp.int32, (1, 256), 1)
    sh = 1
    for _ in range(8):
      rolled = pltpu.roll(acc, sh, axis=1)
      acc = acc + jnp.where(lanes >= sh, rolled, jnp.float32(0.0))
      sh *= 2
    maximum = jnp.max(acc)
    clip_value = (maximum / jnp.float32(100.0)) / jnp.float32(2.0)
    min_g = jnp.sum((acc < clip_value).astype(jnp.float32))
    max_g = jnp.float32(255.0) - jnp.sum(
        (acc >= maximum - clip_value).astype(jnp.float32))
    diff = max_g - min_g
    deg = max_g <= min_g
    safe = jnp.where(deg, jnp.float32(1.0), diff)
    alpha = jnp.where(deg, jnp.float32(1.0), jnp.float32(1.0) / safe)
    beta = jnp.where(deg, jnp.float32(0.0), (-min_g) / safe)
    ab_ref[0] = alpha
    ab_ref[1] = beta

  a = ab_ref[0]
  b = ab_ref[1]
  out_ref[...] = jnp.clip(img_ref[...] * a + b, 0.0, 1.0)


ROWS_PER_BLOCK = 1024
NBLOCKS = (3 * H) // ROWS_PER_BLOCK

_tc_rescale = pl.pallas_call(
    _tc_rescale_body,
    grid=(NBLOCKS,),
    in_specs=[
        pl.BlockSpec((NW, 256), lambda i: (0, 0)),
        pl.BlockSpec((ROWS_PER_BLOCK, W), lambda i: (i, 0)),
    ],
    out_specs=pl.BlockSpec((ROWS_PER_BLOCK, W), lambda i: (i, 0)),
    out_shape=jax.ShapeDtypeStruct((3 * H, W), jnp.float32),
    scratch_shapes=[pltpu.SMEM((2,), jnp.float32)],
    compiler_params=pltpu.CompilerParams(
        dimension_semantics=("arbitrary",)),
)


@jax.jit
def kernel(image_tensor):
  packed = _tc_bins(image_tensor)
  hist_parts = _sc_hist(packed.reshape(NWORDS))
  out = _tc_rescale(hist_parts.reshape(NW, 256),
                    image_tensor.reshape(3 * H, W))
  return out.reshape(3, H, W)
